# baseline (speedup 1.0000x reference)
import functools

import jax
import jax.numpy as jnp
from jax import lax
from jax.experimental import pallas as pl
from jax.experimental.pallas import tpu as pltpu
from jax.experimental.pallas import tpu_sc as plsc

N = 10000
E = 320000
D = 128
DH = D // 2
NC = 2
NS = 16
NW = NC * NS
CH = 128
EPW = E // NW
BLK = 8
ROWS_PW = 80
NBLK = ROWS_PW // BLK
EPAD = ROWS_PW * CH - EPW
NACC = N + 128
NPT = N // NS


def _sc_agg_body(x_hbm, src_hbm, dst_hbm, acc_hbm, src_v, dst_v,
                 rows_a, rows_b, sem_a, sem_b, acc_sh):
    c = lax.axis_index("c")
    s = lax.axis_index("s")
    g = c * NS + s

    z16 = jnp.zeros((16,), jnp.float32)

    @pl.loop(0, CH)
    def _(r):
        for k in range(D // 16):
            rows_a[r, pl.ds(k * 16, 16)] = z16

    for r in range(NPT // CH):
        pltpu.sync_copy(rows_a, acc_sh.at[pl.ds(s * NPT + r * CH, CH)])
    rem = NPT % CH
    pltpu.sync_copy(rows_a.at[pl.ds(0, rem)],
                    acc_sh.at[pl.ds(s * NPT + NPT - rem, rem)])

    @pl.when(s == 0)
    def _():
        pltpu.sync_copy(rows_a.at[pl.ds(0, NACC - N)],
                        acc_sh.at[pl.ds(N, NACC - N)])
    plsc.subcore_barrier()

    def gather(blk, r, buf, sem):
        return pltpu.async_copy(x_hbm.at[blk.at[r]], buf, sem)

    def wait_gather(blk, r, buf, sem):
        pltpu.make_async_copy(x_hbm.at[blk.at[r]], buf, sem).wait()

    def scatter(r, buf):
        pltpu.sync_copy(buf, acc_sh.at[dst_v.at[r]], add=True)

    @pl.loop(0, NBLK)
    def _(b):
        pltpu.sync_copy(src_hbm.at[g, pl.ds(b * BLK, BLK)], src_v)
        pltpu.sync_copy(dst_hbm.at[g, pl.ds(b * BLK, BLK)], dst_v)
        for t in range(BLK):
            gather(src_v, t, rows_a, sem_a).wait()
            scatter(t, rows_a)

    plsc.subcore_barrier()
    pltpu.sync_copy(acc_sh.at[pl.ds(s * NPT, NPT)], acc_hbm.at[c, s])


def _sc_deg_body(dst_hbm, deg_hbm, dst_v, deg_v):
    c = lax.axis_index("c")
    s = lax.axis_index("s")
    g = c * NS + s

    z16 = jnp.zeros((16,), jnp.float32)

    @pl.loop(0, N, step=16)
    def _(i):
        deg_v[pl.ds(i, 16)] = z16

    pltpu.sync_copy(dst_hbm.at[g], dst_v)

    ones16 = jnp.full((16,), 1.0, jnp.float32)

    @pl.loop(0, EPW // 16)
    def _(j):
        plsc.addupdate_scatter(deg_v, [dst_v[j]], ones16)

    pltpu.sync_copy(deg_v, deg_hbm.at[pl.ds(g * N, N)])


@functools.cache
def _sc_kernels():
    mesh = plsc.VectorSubcoreMesh(
        core_axis_name="c", subcore_axis_name="s",
        num_cores=NC, num_subcores=NS)
    params = pltpu.CompilerParams(needs_layout_passes=False)
    agg = pl.kernel(
        _sc_agg_body,
        compiler_params=params,
        out_type=jax.ShapeDtypeStruct((NC, NS, NPT, D), jnp.float32),
        mesh=mesh,
        scratch_types=[
            pltpu.VMEM((BLK, CH), jnp.int32),
            pltpu.VMEM((BLK, CH), jnp.int32),
            pltpu.VMEM((CH, D), jnp.float32),
            pltpu.VMEM((CH, D), jnp.float32),
            pltpu.SemaphoreType.DMA,
            pltpu.SemaphoreType.DMA,
            pltpu.VMEM_SHARED((NACC, D), jnp.float32),
        ],
    )
    deg = pl.kernel(
        _sc_deg_body,
        compiler_params=params,
        out_type=jax.ShapeDtypeStruct((NW * N,), jnp.float32),
        mesh=mesh,
        scratch_types=[
            pltpu.VMEM((EPW // 16, 16), jnp.int32),
            pltpu.VMEM((N,), jnp.float32),
        ],
    )
    return agg, deg


BM = 400
_GRID = N // BM


def _tc_layer_body(relu_out, acc_ref, deg_ref, x_ref, wl_ref, bl_ref, wr_ref,
                   *outs):
    deg = jnp.sum(deg_ref[...], axis=0)
    invd = 1.0 / jnp.maximum(deg, 1.0)
    mean = (acc_ref[0] + acc_ref[1]) * invd
    h1 = (lax.dot_general(mean, wl_ref[...], (((1,), (1,)), ((), ())),
                          preferred_element_type=jnp.float32)
          + bl_ref[...]
          + lax.dot_general(x_ref[...], wr_ref[...], (((1,), (1,)), ((), ())),
                            preferred_element_type=jnp.float32))
    outs[0][...] = h1
    if relu_out:
        outs[1][...] = jnp.maximum(h1, 0.0)


def _make_tc(relu_out):
    n_out = 2 if relu_out else 1
    return pl.pallas_call(
        functools.partial(_tc_layer_body, relu_out),
        grid=(_GRID,),
        in_specs=[
            pl.BlockSpec((NC, BM, D), lambda i: (0, i, 0)),
            pl.BlockSpec((NW, BM, 1), lambda i: (0, i, 0)),
            pl.BlockSpec((BM, D), lambda i: (i, 0)),
            pl.BlockSpec((D, D), lambda i: (0, 0)),
            pl.BlockSpec((1, D), lambda i: (0, 0)),
            pl.BlockSpec((D, D), lambda i: (0, 0)),
        ],
        out_specs=[pl.BlockSpec((BM, D), lambda i: (i, 0))] * n_out,
        out_shape=[jax.ShapeDtypeStruct((N, D), jnp.float32)] * n_out,
    )


_tc_layer_relu = _make_tc(True)
_tc_layer_last = _make_tc(False)


def kernel(x, edge_index, W_l0, b_l0, W_r0, W_l1, b_l1, W_r1):
    sc_agg, sc_deg = _sc_kernels()
    srcf = edge_index[0].astype(jnp.int32).reshape(NW, EPW)
    dstf = edge_index[1].astype(jnp.int32).reshape(NW, EPW)
    pad_src = jnp.zeros((NW, EPAD), jnp.int32)
    pad_dst = jnp.broadcast_to(N + (jnp.arange(EPAD, dtype=jnp.int32) % (NACC - N)),
                               (NW, EPAD))
    src = jnp.concatenate([srcf, pad_src], axis=1).reshape(NW, ROWS_PW, CH)
    dst = jnp.concatenate([dstf, pad_dst], axis=1).reshape(NW, ROWS_PW, CH)
    degp = sc_deg(dstf.reshape(NW, EPW // 16, 16))
    deg = degp.reshape(NW, N, 1)
    acc0 = sc_agg(x, src, dst).reshape(NC, N, D)
    h1, h = _tc_layer_relu(acc0, deg, x, W_l0, b_l0.reshape(1, D), W_r0)
    acc1 = sc_agg(h, src, dst).reshape(NC, N, D)
    (h2,) = _tc_layer_last(acc1, deg, h, W_l1, b_l1.reshape(1, D), W_r1)
    return (h1, h2)


# speedup vs baseline: 1.0149x; 1.0149x over previous
import functools

import jax
import jax.numpy as jnp
from jax import lax
from jax.experimental import pallas as pl
from jax.experimental.pallas import tpu as pltpu
from jax.experimental.pallas import tpu_sc as plsc

N = 10000
E = 320000
D = 128
DH = D // 2
NC = 2
NS = 16
NW = NC * NS
CH = 128
EPW = E // NW
BLK = 8
ROWS_PW = 80
NBLK = ROWS_PW // BLK
EPAD = ROWS_PW * CH - EPW
NACC = N + 128
NPT = N // NS


def _sc_agg_body(x_hbm, src_hbm, dst_hbm, acc_hbm, src_v, dst_v,
                 rows_a, sem_a, acc_sh):
    c = lax.axis_index("c")
    s = lax.axis_index("s")
    g = c * NS + s

    z16 = jnp.zeros((16,), jnp.float32)

    @pl.loop(0, CH)
    def _(r):
        for k in range(D // 16):
            rows_a[r, pl.ds(k * 16, 16)] = z16

    for r in range(NPT // CH):
        pltpu.sync_copy(rows_a, acc_sh.at[pl.ds(s * NPT + r * CH, CH)])
    rem = NPT % CH
    pltpu.sync_copy(rows_a.at[pl.ds(0, rem)],
                    acc_sh.at[pl.ds(s * NPT + NPT - rem, rem)])

    @pl.when(s == 0)
    def _():
        pltpu.sync_copy(rows_a.at[pl.ds(0, NACC - N)],
                        acc_sh.at[pl.ds(N, NACC - N)])
    plsc.subcore_barrier()

    def gather(blk, r, buf, sem):
        return pltpu.async_copy(x_hbm.at[blk.at[r]], buf, sem)

    def wait_gather(blk, r, buf, sem):
        pltpu.make_async_copy(x_hbm.at[blk.at[r]], buf, sem).wait()

    def scatter(r, buf):
        pltpu.sync_copy(buf, acc_sh.at[dst_v.at[r]], add=True)

    pltpu.sync_copy(src_hbm.at[g], src_v)
    pltpu.sync_copy(dst_hbm.at[g], dst_v)

    @pl.loop(0, ROWS_PW)
    def _(j):
        gather(src_v, j, rows_a, sem_a).wait()
        scatter(j, rows_a)

    plsc.subcore_barrier()
    pltpu.sync_copy(acc_sh.at[pl.ds(s * NPT, NPT)], acc_hbm.at[c, s])


def _sc_deg_body(dst_hbm, deg_hbm, dst_v, deg_v):
    c = lax.axis_index("c")
    s = lax.axis_index("s")
    g = c * NS + s

    z16 = jnp.zeros((16,), jnp.float32)

    @pl.loop(0, N, step=16)
    def _(i):
        deg_v[pl.ds(i, 16)] = z16

    pltpu.sync_copy(dst_hbm.at[g], dst_v)

    ones16 = jnp.full((16,), 1.0, jnp.float32)

    @pl.loop(0, EPW // 16)
    def _(j):
        plsc.addupdate_scatter(deg_v, [dst_v[j]], ones16)

    pltpu.sync_copy(deg_v, deg_hbm.at[pl.ds(g * N, N)])


@functools.cache
def _sc_kernels():
    mesh = plsc.VectorSubcoreMesh(
        core_axis_name="c", subcore_axis_name="s",
        num_cores=NC, num_subcores=NS)
    params = pltpu.CompilerParams(needs_layout_passes=False)
    agg = pl.kernel(
        _sc_agg_body,
        compiler_params=params,
        out_type=jax.ShapeDtypeStruct((NC, NS, NPT, D), jnp.float32),
        mesh=mesh,
        scratch_types=[
            pltpu.VMEM((ROWS_PW, CH), jnp.int32),
            pltpu.VMEM((ROWS_PW, CH), jnp.int32),
            pltpu.VMEM((CH, D), jnp.float32),
            pltpu.SemaphoreType.DMA,
            pltpu.VMEM_SHARED((NACC, D), jnp.float32),
        ],
    )
    deg = pl.kernel(
        _sc_deg_body,
        compiler_params=params,
        out_type=jax.ShapeDtypeStruct((NW * N,), jnp.float32),
        mesh=mesh,
        scratch_types=[
            pltpu.VMEM((EPW // 16, 16), jnp.int32),
            pltpu.VMEM((N,), jnp.float32),
        ],
    )
    return agg, deg


BM = 400
_GRID = N // BM


def _tc_layer_body(relu_out, acc_ref, deg_ref, x_ref, wl_ref, bl_ref, wr_ref,
                   *outs):
    deg = jnp.sum(deg_ref[...], axis=0)
    invd = 1.0 / jnp.maximum(deg, 1.0)
    mean = (acc_ref[0] + acc_ref[1]) * invd
    h1 = (lax.dot_general(mean, wl_ref[...], (((1,), (1,)), ((), ())),
                          preferred_element_type=jnp.float32)
          + bl_ref[...]
          + lax.dot_general(x_ref[...], wr_ref[...], (((1,), (1,)), ((), ())),
                            preferred_element_type=jnp.float32))
    outs[0][...] = h1
    if relu_out:
        outs[1][...] = jnp.maximum(h1, 0.0)


def _make_tc(relu_out):
    n_out = 2 if relu_out else 1
    return pl.pallas_call(
        functools.partial(_tc_layer_body, relu_out),
        grid=(_GRID,),
        in_specs=[
            pl.BlockSpec((NC, BM, D), lambda i: (0, i, 0)),
            pl.BlockSpec((NW, BM, 1), lambda i: (0, i, 0)),
            pl.BlockSpec((BM, D), lambda i: (i, 0)),
            pl.BlockSpec((D, D), lambda i: (0, 0)),
            pl.BlockSpec((1, D), lambda i: (0, 0)),
            pl.BlockSpec((D, D), lambda i: (0, 0)),
        ],
        out_specs=[pl.BlockSpec((BM, D), lambda i: (i, 0))] * n_out,
        out_shape=[jax.ShapeDtypeStruct((N, D), jnp.float32)] * n_out,
    )


_tc_layer_relu = _make_tc(True)
_tc_layer_last = _make_tc(False)


def kernel(x, edge_index, W_l0, b_l0, W_r0, W_l1, b_l1, W_r1):
    sc_agg, sc_deg = _sc_kernels()
    srcf = edge_index[0].astype(jnp.int32).reshape(NW, EPW)
    dstf = edge_index[1].astype(jnp.int32).reshape(NW, EPW)
    pad_src = jnp.zeros((NW, EPAD), jnp.int32)
    pad_dst = jnp.broadcast_to(N + (jnp.arange(EPAD, dtype=jnp.int32) % (NACC - N)),
                               (NW, EPAD))
    src = jnp.concatenate([srcf, pad_src], axis=1).reshape(NW, ROWS_PW, CH)
    dst = jnp.concatenate([dstf, pad_dst], axis=1).reshape(NW, ROWS_PW, CH)
    degp = sc_deg(dstf.reshape(NW, EPW // 16, 16))
    deg = degp.reshape(NW, N, 1)
    acc0 = sc_agg(x, src, dst).reshape(NC, N, D)
    h1, h = _tc_layer_relu(acc0, deg, x, W_l0, b_l0.reshape(1, D), W_r0)
    acc1 = sc_agg(h, src, dst).reshape(NC, N, D)
    (h2,) = _tc_layer_last(acc1, deg, h, W_l1, b_l1.reshape(1, D), W_r1)
    return (h1, h2)


# speedup vs baseline: 1.9821x; 1.9529x over previous
import functools

import jax
import jax.numpy as jnp
from jax import lax
from jax.experimental import pallas as pl
from jax.experimental.pallas import tpu as pltpu
from jax.experimental.pallas import tpu_sc as plsc

N = 10000
E = 320000
D = 128
DH = D // 2
NC = 2
NS = 16
NW = NC * NS
CH = 80
EPW = E // NW
ROWS_PW = EPW // CH
SRC_SHIFT = 14
NPT = N // NS


def _sc_agg_body(x_hbm, pk_hbm, acc_hbm, pk_v, src_b, dst_b,
                 rows_a, rows_b, sem_a, sem_b, acc_sh):
    c = lax.axis_index("c")
    s = lax.axis_index("s")
    g = c * NS + s

    z16 = jnp.zeros((16,), jnp.float32)

    @pl.loop(0, CH)
    def _(r):
        for k in range(D // 16):
            rows_a[r, pl.ds(k * 16, 16)] = z16

    for r in range(NPT // CH):
        pltpu.sync_copy(rows_a, acc_sh.at[pl.ds(s * NPT + r * CH, CH)])
    rem = NPT % CH
    pltpu.sync_copy(rows_a.at[pl.ds(0, rem)],
                    acc_sh.at[pl.ds(s * NPT + NPT - rem, rem)])
    plsc.subcore_barrier()

    pltpu.sync_copy(pk_hbm.at[g], pk_v)

    def unpack(j, r):
        for k in range(CH // 16):
            p = pk_v[j, pl.ds(k * 16, 16)]
            src_b[r, pl.ds(k * 16, 16)] = p >> SRC_SHIFT
            dst_b[r, pl.ds(k * 16, 16)] = p & ((1 << SRC_SHIFT) - 1)

    def gather(r, buf, sem):
        return pltpu.async_copy(x_hbm.at[src_b.at[r]], buf, sem)

    def wait_gather(r, buf, sem):
        pltpu.make_async_copy(x_hbm.at[src_b.at[r]], buf, sem).wait()

    def scatter(r, buf):
        pltpu.sync_copy(buf, acc_sh.at[dst_b.at[r]], add=True)

    unpack(0, 0)
    gather(0, rows_a, sem_a)

    @pl.loop(0, (ROWS_PW - 3) // 2)
    def _(t):
        j = 2 * t
        unpack(j + 1, 1)
        wait_gather(0, rows_a, sem_a)
        gather(1, rows_b, sem_b)
        scatter(0, rows_a)
        unpack(j + 2, 0)
        wait_gather(1, rows_b, sem_b)
        gather(0, rows_a, sem_a)
        scatter(1, rows_b)

    jf = ROWS_PW - 3
    unpack(jf + 1, 1)
    wait_gather(0, rows_a, sem_a)
    gather(1, rows_b, sem_b)
    scatter(0, rows_a)
    unpack(jf + 2, 0)
    wait_gather(1, rows_b, sem_b)
    gather(0, rows_a, sem_a)
    scatter(1, rows_b)
    wait_gather(0, rows_a, sem_a)
    scatter(0, rows_a)

    plsc.subcore_barrier()
    pltpu.sync_copy(acc_sh.at[pl.ds(s * NPT, NPT)], acc_hbm.at[c, s])


def _sc_deg_body(dst_hbm, deg_hbm, dst_v, deg_v):
    c = lax.axis_index("c")
    s = lax.axis_index("s")
    g = c * NS + s

    z16 = jnp.zeros((16,), jnp.float32)

    @pl.loop(0, N, step=16)
    def _(i):
        deg_v[pl.ds(i, 16)] = z16

    pltpu.sync_copy(dst_hbm.at[g], dst_v)

    ones16 = jnp.full((16,), 1.0, jnp.float32)

    @pl.loop(0, EPW // 16)
    def _(j):
        plsc.addupdate_scatter(deg_v, [dst_v[j]], ones16)

    pltpu.sync_copy(deg_v, deg_hbm.at[pl.ds(g * N, N)])


@functools.cache
def _sc_kernels():
    mesh = plsc.VectorSubcoreMesh(
        core_axis_name="c", subcore_axis_name="s",
        num_cores=NC, num_subcores=NS)
    params = pltpu.CompilerParams(needs_layout_passes=False)
    agg = pl.kernel(
        _sc_agg_body,
        compiler_params=params,
        out_type=jax.ShapeDtypeStruct((NC, NS, NPT, D), jnp.float32),
        mesh=mesh,
        scratch_types=[
            pltpu.VMEM((ROWS_PW, CH), jnp.int32),
            pltpu.VMEM((8, CH), jnp.int32),
            pltpu.VMEM((8, CH), jnp.int32),
            pltpu.VMEM((CH, D), jnp.float32),
            pltpu.VMEM((CH, D), jnp.float32),
            pltpu.SemaphoreType.DMA,
            pltpu.SemaphoreType.DMA,
            pltpu.VMEM_SHARED((N, D), jnp.float32),
        ],
    )
    deg = pl.kernel(
        _sc_deg_body,
        compiler_params=params,
        out_type=jax.ShapeDtypeStruct((NW * N,), jnp.float32),
        mesh=mesh,
        scratch_types=[
            pltpu.VMEM((EPW // 16, 16), jnp.int32),
            pltpu.VMEM((N,), jnp.float32),
        ],
    )
    return agg, deg


BM = 400
_GRID = N // BM


def _tc_layer_body(relu_out, acc_ref, deg_ref, x_ref, wl_ref, bl_ref, wr_ref,
                   *outs):
    deg = jnp.sum(deg_ref[...], axis=0)
    invd = 1.0 / jnp.maximum(deg, 1.0)
    mean = (acc_ref[0] + acc_ref[1]) * invd
    h1 = (lax.dot_general(mean, wl_ref[...], (((1,), (1,)), ((), ())),
                          preferred_element_type=jnp.float32)
          + bl_ref[...]
          + lax.dot_general(x_ref[...], wr_ref[...], (((1,), (1,)), ((), ())),
                            preferred_element_type=jnp.float32))
    outs[0][...] = h1
    if relu_out:
        outs[1][...] = jnp.maximum(h1, 0.0)


def _make_tc(relu_out):
    n_out = 2 if relu_out else 1
    return pl.pallas_call(
        functools.partial(_tc_layer_body, relu_out),
        grid=(_GRID,),
        in_specs=[
            pl.BlockSpec((NC, BM, D), lambda i: (0, i, 0)),
            pl.BlockSpec((NW, BM, 1), lambda i: (0, i, 0)),
            pl.BlockSpec((BM, D), lambda i: (i, 0)),
            pl.BlockSpec((D, D), lambda i: (0, 0)),
            pl.BlockSpec((1, D), lambda i: (0, 0)),
            pl.BlockSpec((D, D), lambda i: (0, 0)),
        ],
        out_specs=[pl.BlockSpec((BM, D), lambda i: (i, 0))] * n_out,
        out_shape=[jax.ShapeDtypeStruct((N, D), jnp.float32)] * n_out,
    )


_tc_layer_relu = _make_tc(True)
_tc_layer_last = _make_tc(False)


def kernel(x, edge_index, W_l0, b_l0, W_r0, W_l1, b_l1, W_r1):
    sc_agg, sc_deg = _sc_kernels()
    srcf = edge_index[0].astype(jnp.int32)
    dstf = edge_index[1].astype(jnp.int32)
    pk = ((srcf << SRC_SHIFT) | dstf).reshape(NW, ROWS_PW, CH)
    degp = sc_deg(dstf.reshape(NW, EPW // 16, 16))
    deg = degp.reshape(NW, N, 1)
    acc0 = sc_agg(x, pk).reshape(NC, N, D)
    h1, h = _tc_layer_relu(acc0, deg, x, W_l0, b_l0.reshape(1, D), W_r0)
    acc1 = sc_agg(h, pk).reshape(NC, N, D)
    (h2,) = _tc_layer_last(acc1, deg, h, W_l1, b_l1.reshape(1, D), W_r1)
    return (h1, h2)
